# Initial kernel scaffold; baseline (speedup 1.0000x reference)
import jax
import jax.numpy as jnp
from jax.experimental import pallas as pl


def kernel(x, edge_index, batch, W1, b1, W2, b2, W3, b3, fc_W, fc_b):
    raise NotImplementedError("write your pallas kernel here")


# speedup vs baseline: 9.6565x; 9.6565x over previous
import jax
import jax.numpy as jnp
from jax import lax
from jax.experimental import pallas as pl
from jax.experimental.pallas import tpu as pltpu
from jax.experimental.pallas import tpu_sc as plsc

N = 10000
F = 128
HF = F // 2
NG = 64
E = 320000
NC = 2
NS = 16
NW = NC * NS
CH = 128
NCH = 79
EPT = NCH * CH
PAD_E = NW * EPT - E
N_PAD = 10240
RPT = N_PAD // NS

_MESH = plsc.VectorSubcoreMesh(core_axis_name="c", subcore_axis_name="s",
                               num_cores=NC, num_subcores=NS)


def _zero_1d(ref, n):
    def body(i, c):
        ref[pl.ds(i * 16, 16)] = jnp.zeros((16,), jnp.float32)
        return c
    lax.fori_loop(0, n // 16, body, 0)


def _zero_2d(ref, nrows, ncols):
    def body(i, c):
        for k in range(ncols // 16):
            ref[i, pl.ds(k * 16, 16)] = jnp.zeros((16,), jnp.float32)
        return c
    lax.fori_loop(0, nrows, body, 0)


def _sc_deg_body(dst_hbm, deg0, deg1, dst_v, ones_v, acc):
    c = lax.axis_index("c")
    s = lax.axis_index("s")
    wid = s * NC + c
    for k in range(CH // 16):
        ones_v[pl.ds(k * 16, 16)] = jnp.ones((16,), jnp.float32)
    _zero_1d(ones_v.at[pl.ds(CH, CH)], CH)
    def zbody(i, cr):
        pltpu.sync_copy(ones_v.at[pl.ds(CH, CH)],
                        acc.at[pl.ds(s * RPT + i * CH, CH)])
        return cr
    lax.fori_loop(0, RPT // CH, zbody, 0)
    pltpu.sync_copy(dst_hbm.at[wid], dst_v)
    plsc.subcore_barrier()

    def body(j, carry):
        pltpu.sync_copy(ones_v.at[pl.ds(0, CH)], acc.at[dst_v.at[j]], add=True)
        return carry

    lax.fori_loop(0, NCH, body, 0)
    plsc.subcore_barrier()

    @pl.when(c == 0)
    def _():
        pltpu.sync_copy(acc.at[pl.ds(s * RPT, RPT)], deg0.at[pl.ds(s * RPT, RPT)])

    @pl.when(c == 1)
    def _():
        pltpu.sync_copy(acc.at[pl.ds(s * RPT, RPT)], deg1.at[pl.ds(s * RPT, RPT)])


_sc_deg = pl.kernel(
    _sc_deg_body,
    out_type=(jax.ShapeDtypeStruct((N_PAD,), jnp.float32),
              jax.ShapeDtypeStruct((N_PAD,), jnp.float32)),
    mesh=_MESH,
    scratch_types=[
        pltpu.VMEM((NCH, CH), jnp.int32),
        pltpu.VMEM((2 * CH,), jnp.float32),
        pltpu.VMEM_SHARED((N_PAD,), jnp.float32),
    ],
)


def _sc_scatter_body(ga_hbm, gb_hbm, src_hbm, dst_hbm, o0a, o1a, o0b, o1b,
                     src_v, dst_v, rows_v, zb, acc, sem):
    c = lax.axis_index("c")
    s = lax.axis_index("s")
    wid = s * NC + c
    _zero_2d(zb, CH, HF)

    def zero_acc():
        def zbody(i, cr):
            pltpu.sync_copy(zb, acc.at[pl.ds(s * RPT + i * CH, CH)])
            return cr
        lax.fori_loop(0, RPT // CH, zbody, 0)

    zero_acc()
    pltpu.sync_copy(src_hbm.at[wid], src_v)
    pltpu.sync_copy(dst_hbm.at[wid], dst_v)
    plsc.subcore_barrier()

    def scatter_phase(g_hbm):
        def body(j, carry):
            pltpu.async_copy(g_hbm.at[src_v.at[j]], rows_v, sem).wait()
            pltpu.sync_copy(rows_v, acc.at[dst_v.at[j]], add=True)
            return carry
        lax.fori_loop(0, NCH, body, 0)

    scatter_phase(ga_hbm)
    plsc.subcore_barrier()

    @pl.when(c == 0)
    def _():
        pltpu.sync_copy(acc.at[pl.ds(s * RPT, RPT)], o0a.at[pl.ds(s * RPT, RPT)])

    @pl.when(c == 1)
    def _():
        pltpu.sync_copy(acc.at[pl.ds(s * RPT, RPT)], o1a.at[pl.ds(s * RPT, RPT)])

    zero_acc()
    plsc.subcore_barrier()

    scatter_phase(gb_hbm)
    plsc.subcore_barrier()

    @pl.when(c == 0)
    def _():
        pltpu.sync_copy(acc.at[pl.ds(s * RPT, RPT)], o0b.at[pl.ds(s * RPT, RPT)])

    @pl.when(c == 1)
    def _():
        pltpu.sync_copy(acc.at[pl.ds(s * RPT, RPT)], o1b.at[pl.ds(s * RPT, RPT)])


_sc_scatter = pl.kernel(
    _sc_scatter_body,
    out_type=tuple(jax.ShapeDtypeStruct((N_PAD, HF), jnp.float32)
                   for _ in range(4)),
    mesh=_MESH,
    scratch_types=[
        pltpu.VMEM((NCH, CH), jnp.int32),
        pltpu.VMEM((NCH, CH), jnp.int32),
        pltpu.VMEM((CH, HF), jnp.float32),
        pltpu.VMEM((CH, HF), jnp.float32),
        pltpu.VMEM_SHARED((N_PAD, HF), jnp.float32),
        pltpu.SemaphoreType.DMA,
    ],
    compiler_params=pltpu.CompilerParams(use_tc_tiling_on_sc=False),
)


def _tc1_body(x_ref, w_ref, d0_ref, d1_ref, ga_ref, gb_ref):
    dinv = lax.rsqrt(d0_ref[...] + d1_ref[...] + 1.0)
    g = dinv * jnp.dot(x_ref[...], w_ref[...], preferred_element_type=jnp.float32)
    ga_ref[...] = g[:, :HF]
    gb_ref[...] = g[:, HF:]


_tc1 = pl.pallas_call(
    _tc1_body,
    out_shape=(jax.ShapeDtypeStruct((N, HF), jnp.float32),
               jax.ShapeDtypeStruct((N, HF), jnp.float32)),
)


def _tc_mid_body(p0a, p1a, p0b, p1b, gpa, gpb, b_ref, w_ref, d0_ref, d1_ref,
                 ga_ref, gb_ref):
    dinv = lax.rsqrt(d0_ref[...] + d1_ref[...] + 1.0)
    xa = p0a[...] + p1a[...] + gpa[...]
    xb = p0b[...] + p1b[...] + gpb[...]
    xn = jnp.maximum(dinv * jnp.concatenate([xa, xb], axis=1) + b_ref[...], 0.0)
    g = dinv * jnp.dot(xn, w_ref[...], preferred_element_type=jnp.float32)
    ga_ref[...] = g[:, :HF]
    gb_ref[...] = g[:, HF:]


_tc_mid = pl.pallas_call(
    _tc_mid_body,
    out_shape=(jax.ShapeDtypeStruct((N, HF), jnp.float32),
               jax.ShapeDtypeStruct((N, HF), jnp.float32)),
)


def _tc_final_body(p0a, p1a, p0b, p1b, gpa, gpb, b_ref, d0_ref, d1_ref,
                   fcw_ref, fcb_ref, batch_ref, out_ref):
    dinv = lax.rsqrt(d0_ref[...] + d1_ref[...] + 1.0)
    xa = p0a[...] + p1a[...] + gpa[...]
    xb = p0b[...] + p1b[...] + gpb[...]
    y = jnp.maximum(dinv * jnp.concatenate([xa, xb], axis=1) + b_ref[...], 0.0)
    z = jnp.sum(y * fcw_ref[...], axis=1, keepdims=True)
    gid = lax.broadcasted_iota(jnp.int32, (1, NG), 1)
    eq = (batch_ref[...] == gid).astype(jnp.float32)
    ssum = jnp.sum(eq * z, axis=0)
    cnt = jnp.sum(eq, axis=0)
    out_ref[...] = (ssum / jnp.maximum(cnt, 1.0))[:, None] + fcb_ref[...]


_tc_final = pl.pallas_call(
    _tc_final_body,
    out_shape=jax.ShapeDtypeStruct((NG, 1), jnp.float32),
)


def kernel(x, edge_index, batch, W1, b1, W2, b2, W3, b3, fc_W, fc_b):
    src = edge_index[0]
    dst = edge_index[1]
    src_p = jnp.concatenate([src, jnp.zeros((PAD_E,), jnp.int32)]).reshape(NW, NCH, CH)
    dst_p = jnp.concatenate([dst, jnp.full((PAD_E,), N, jnp.int32)]).reshape(NW, NCH, CH)

    deg0, deg1 = _sc_deg(dst_p)
    d0 = deg0[:N].reshape(N, 1)
    d1 = deg1[:N].reshape(N, 1)

    ga, gb = _tc1(x, W1, d0, d1)
    p = _sc_scatter(ga, gb, src_p, dst_p)
    ga, gb = _tc_mid(p[0][:N], p[1][:N], p[2][:N], p[3][:N], ga, gb,
                     b1.reshape(1, F), W2, d0, d1)
    p = _sc_scatter(ga, gb, src_p, dst_p)
    ga, gb = _tc_mid(p[0][:N], p[1][:N], p[2][:N], p[3][:N], ga, gb,
                     b2.reshape(1, F), W3, d0, d1)
    p = _sc_scatter(ga, gb, src_p, dst_p)
    return _tc_final(p[0][:N], p[1][:N], p[2][:N], p[3][:N], ga, gb,
                     b3.reshape(1, F), d0, d1,
                     fc_W.reshape(1, F), fc_b.reshape(1, 1),
                     batch.reshape(N, 1))
